# baseline (speedup 1.0000x reference)
import functools

import jax
import jax.numpy as jnp
from jax import lax
from jax.experimental import pallas as pl
from jax.experimental.pallas import tpu as pltpu, tpu_sc as plsc

EPS = 1e-06
MEAN_READS = 5000.0
LN2 = 0.6931471805599453
SQRT2 = 1.4142135381698608

N_ROWS = 16384
R = 3
NC, NS, L = 2, 16, 16
NW = NC * NS
CHUNK = N_ROWS // NW
VECS = CHUNK // L


def _ln(x):
    bits = lax.bitcast_convert_type(x, jnp.int32)
    e = ((bits >> 23) & 0xFF) - 127
    m = lax.bitcast_convert_type((bits & 0x7FFFFF) | (127 << 23), jnp.float32)
    big = m > SQRT2
    m = jnp.where(big, m * 0.5, m)
    e = e + big.astype(jnp.int32)
    t = (m - 1.0) / (m + 1.0)
    z = t * t
    ln_m = 2.0 * t * (1.0 + z * (1.0 / 3.0 + z * (1.0 / 5.0 + z * (1.0 / 7.0 + z * (1.0 / 9.0)))))
    out = e.astype(jnp.float32) * LN2 + ln_m
    bad = jnp.logical_not(x < jnp.inf)
    return jnp.where(bad, x, out)


def _sc_body(gene_hbm, dsr_hbm, tor_hbm, tf_hbm,
             o0_hbm, o1_hbm, o2_hbm,
             idx_v, i0_v, i1_v, i2_v, p0_v, p1_v, p2_v, dsr_v, tor_v,
             sem0, sem12):
    wid = lax.axis_index("s") * NC + lax.axis_index("c")
    base = wid * CHUNK
    pltpu.sync_copy(gene_hbm.at[pl.ds(base, CHUNK)], idx_v)
    for k in range(VECS):
        s = pl.ds(k * L, L)
        i0_v[s] = idx_v[s] * 3
    g0 = pltpu.async_copy(tf_hbm.at[i0_v], p0_v, sem0)
    for k in range(VECS):
        s = pl.ds(k * L, L)
        i1_v[s] = i0_v[s] + 1
        i2_v[s] = i0_v[s] + 2
    g1 = pltpu.async_copy(tf_hbm.at[i1_v], p1_v, sem12)
    g2 = pltpu.async_copy(tf_hbm.at[i2_v], p2_v, sem12)
    pltpu.sync_copy(dsr_hbm.at[pl.ds(base, CHUNK)], dsr_v)
    pltpu.sync_copy(tor_hbm.at[pl.ds(base, CHUNK)], tor_v)
    g0.wait()
    for k in range(VECS):
        s = pl.ds(k * L, L)
        scale = tor_v[s] / (MEAN_READS * dsr_v[s])
        p0_v[s] = p0_v[s] + _ln(EPS + scale)
    pltpu.sync_copy(p0_v, o0_hbm.at[pl.ds(base, CHUNK)])
    g1.wait()
    g2.wait()
    pltpu.sync_copy(p1_v, o1_hbm.at[pl.ds(base, CHUNK)])
    pltpu.sync_copy(p2_v, o2_hbm.at[pl.ds(base, CHUNK)])


@jax.jit
def _sc_call(gene_idx, dsr, tor, table):
    mesh = plsc.VectorSubcoreMesh(core_axis_name="c", subcore_axis_name="s")
    plane = jax.ShapeDtypeStruct((N_ROWS,), jnp.float32)
    fn = functools.partial(
        pl.kernel,
        out_type=(plane, plane, plane),
        mesh=mesh,
        scratch_types=[
            pltpu.VMEM((CHUNK,), jnp.int32),
            pltpu.VMEM((CHUNK,), jnp.int32),
            pltpu.VMEM((CHUNK,), jnp.int32),
            pltpu.VMEM((CHUNK,), jnp.int32),
            pltpu.VMEM((CHUNK,), jnp.float32),
            pltpu.VMEM((CHUNK,), jnp.float32),
            pltpu.VMEM((CHUNK,), jnp.float32),
            pltpu.VMEM((CHUNK,), jnp.float32),
            pltpu.VMEM((CHUNK,), jnp.float32),
            pltpu.SemaphoreType.DMA,
            pltpu.SemaphoreType.DMA,
        ],
        compiler_params=pltpu.CompilerParams(use_tc_tiling_on_sc=False,
                                             needs_layout_passes=False),
    )(_sc_body)
    o0, o1, o2 = fn(gene_idx, dsr, tor, table.reshape(-1))
    return jnp.stack([o0, o1, o2], axis=1)


def kernel(gene_index_tensor_n, cell_index_tensor_n, downsampling_rate_tensor_n,
           total_obs_reads_per_cell_tensor_n, cell_features_nf, readout_bias_gr):
    return _sc_call(gene_index_tensor_n.astype(jnp.int32),
                    downsampling_rate_tensor_n,
                    total_obs_reads_per_cell_tensor_n,
                    readout_bias_gr)


# speedup vs baseline: 2.0178x; 2.0178x over previous
import functools

import jax
import jax.numpy as jnp
from jax import lax
from jax.experimental import pallas as pl
from jax.experimental.pallas import tpu as pltpu, tpu_sc as plsc

EPS = 1e-06
MEAN_READS = 5000.0
LN2 = 0.6931471805599453
SQRT2 = 1.4142135381698608

N_ROWS = 16384
R = 3
NC, NS, L = 2, 16, 16
NW = NC * NS
CHUNK = N_ROWS // NW
VECS = CHUNK // L


def _ln(x):
    bits = lax.bitcast_convert_type(x, jnp.int32)
    e = ((bits >> 23) & 0xFF) - 127
    m = lax.bitcast_convert_type((bits & 0x7FFFFF) | (127 << 23), jnp.float32)
    big = m > SQRT2
    m = jnp.where(big, m * 0.5, m)
    e = e + big.astype(jnp.int32)
    t = (m - 1.0) / (m + 1.0)
    z = t * t
    ln_m = 2.0 * t * (1.0 + z * (1.0 / 3.0 + z * (1.0 / 5.0 + z * (1.0 / 7.0 + z * (1.0 / 9.0)))))
    out = e.astype(jnp.float32) * LN2 + ln_m
    bad = jnp.logical_not(x < jnp.inf)
    return jnp.where(bad, x, out)


def _sc_body(gene_hbm, dsr_hbm, tor_hbm, c0_hbm, c1_hbm, c2_hbm, out_hbm,
             idx_v, p0_v, p1_v, p2_v, dsr_v, tor_v, rows_v, sem0, sem12):
    wid = lax.axis_index("s") * NC + lax.axis_index("c")
    base = wid * CHUNK
    pltpu.sync_copy(gene_hbm.at[pl.ds(base, CHUNK)], idx_v)
    g0 = pltpu.async_copy(c0_hbm.at[idx_v], p0_v, sem0)
    g1 = pltpu.async_copy(c1_hbm.at[idx_v], p1_v, sem12)
    g2 = pltpu.async_copy(c2_hbm.at[idx_v], p2_v, sem12)
    pltpu.sync_copy(dsr_hbm.at[pl.ds(base, CHUNK)], dsr_v)
    pltpu.sync_copy(tor_hbm.at[pl.ds(base, CHUNK)], tor_v)
    lane = lax.iota(jnp.int32, L)
    col0 = jnp.zeros((L,), jnp.int32)
    col1 = col0 + 1
    col2 = col0 + 2
    g0.wait()
    for k in range(VECS):
        s = pl.ds(k * L, L)
        rid = lane + (k * L)
        scale = tor_v[s] / (MEAN_READS * dsr_v[s])
        plsc.store_scatter(rows_v, [rid, col0], p0_v[s] + _ln(EPS + scale))
    g1.wait()
    g2.wait()
    for k in range(VECS):
        s = pl.ds(k * L, L)
        rid = lane + (k * L)
        plsc.store_scatter(rows_v, [rid, col1], p1_v[s])
        plsc.store_scatter(rows_v, [rid, col2], p2_v[s])
    pltpu.sync_copy(rows_v, out_hbm.at[pl.ds(base, CHUNK)])


@jax.jit
def _sc_call(gene_idx, dsr, tor, table):
    mesh = plsc.VectorSubcoreMesh(core_axis_name="c", subcore_axis_name="s")
    fn = functools.partial(
        pl.kernel,
        out_type=jax.ShapeDtypeStruct((N_ROWS, R), jnp.float32),
        mesh=mesh,
        scratch_types=[
            pltpu.VMEM((CHUNK,), jnp.int32),
            pltpu.VMEM((CHUNK,), jnp.float32),
            pltpu.VMEM((CHUNK,), jnp.float32),
            pltpu.VMEM((CHUNK,), jnp.float32),
            pltpu.VMEM((CHUNK,), jnp.float32),
            pltpu.VMEM((CHUNK,), jnp.float32),
            pltpu.VMEM((CHUNK, R), jnp.float32),
            pltpu.SemaphoreType.DMA,
            pltpu.SemaphoreType.DMA,
        ],
        compiler_params=pltpu.CompilerParams(use_tc_tiling_on_sc=False,
                                             needs_layout_passes=False),
    )(_sc_body)
    c0, c1, c2 = table[:, 0], table[:, 1], table[:, 2]
    return fn(gene_idx, dsr, tor, c0, c1, c2)


def kernel(gene_index_tensor_n, cell_index_tensor_n, downsampling_rate_tensor_n,
           total_obs_reads_per_cell_tensor_n, cell_features_nf, readout_bias_gr):
    return _sc_call(gene_index_tensor_n.astype(jnp.int32),
                    downsampling_rate_tensor_n,
                    total_obs_reads_per_cell_tensor_n,
                    readout_bias_gr)


# speedup vs baseline: 3.3119x; 1.6413x over previous
import functools

import jax
import jax.numpy as jnp
from jax import lax
from jax.experimental import pallas as pl
from jax.experimental.pallas import tpu as pltpu, tpu_sc as plsc

EPS = 1e-06
MEAN_READS = 5000.0
LN2 = 0.6931471805599453
SQRT2 = 1.4142135381698608

N_ROWS = 16384
R = 3
NC, NS, L = 2, 16, 16
NW = NC * NS
CHUNK = N_ROWS // NW
VECS = CHUNK // L


def _ln(x):
    bits = lax.bitcast_convert_type(x, jnp.int32)
    e = ((bits >> 23) & 0xFF) - 127
    m = lax.bitcast_convert_type((bits & 0x7FFFFF) | (127 << 23), jnp.float32)
    big = m > SQRT2
    m = jnp.where(big, m * 0.5, m)
    e = e + big.astype(jnp.int32)
    t = (m - 1.0) / (m + 1.0)
    z = t * t
    ln_m = 2.0 * t * (1.0 + z * (1.0 / 3.0 + z * (1.0 / 5.0 + z * (1.0 / 7.0 + z * (1.0 / 9.0)))))
    out = e.astype(jnp.float32) * LN2 + ln_m
    bad = jnp.logical_not(x < jnp.inf)
    return jnp.where(bad, x, out)


def _sc_body(gene_hbm, dsr_hbm, tor_hbm, tt_hbm,
             o0_hbm, o1_hbm, o2_hbm,
             idx_v, p0_v, p1_v, p2_v, dsr_v, tor_v, sem0, sem12):
    wid = lax.axis_index("s") * NC + lax.axis_index("c")
    base = wid * CHUNK
    pltpu.sync_copy(gene_hbm.at[pl.ds(base, CHUNK)], idx_v)
    g0 = pltpu.async_copy(tt_hbm.at[0].at[idx_v], p0_v, sem0)
    g1 = pltpu.async_copy(tt_hbm.at[1].at[idx_v], p1_v, sem12)
    g2 = pltpu.async_copy(tt_hbm.at[2].at[idx_v], p2_v, sem12)
    pltpu.sync_copy(dsr_hbm.at[pl.ds(base, CHUNK)], dsr_v)
    pltpu.sync_copy(tor_hbm.at[pl.ds(base, CHUNK)], tor_v)
    g0.wait()
    for k in range(VECS):
        s = pl.ds(k * L, L)
        scale = tor_v[s] / (MEAN_READS * dsr_v[s])
        p0_v[s] = p0_v[s] + _ln(EPS + scale)
    pltpu.sync_copy(p0_v, o0_hbm.at[pl.ds(base, CHUNK)])
    g1.wait()
    g2.wait()
    pltpu.sync_copy(p1_v, o1_hbm.at[pl.ds(base, CHUNK)])
    pltpu.sync_copy(p2_v, o2_hbm.at[pl.ds(base, CHUNK)])


@jax.jit
def _sc_call(gene_idx, dsr, tor, table):
    mesh = plsc.VectorSubcoreMesh(core_axis_name="c", subcore_axis_name="s")
    plane = jax.ShapeDtypeStruct((N_ROWS,), jnp.float32)
    fn = functools.partial(
        pl.kernel,
        out_type=(plane, plane, plane),
        mesh=mesh,
        scratch_types=[
            pltpu.VMEM((CHUNK,), jnp.int32),
            pltpu.VMEM((CHUNK,), jnp.float32),
            pltpu.VMEM((CHUNK,), jnp.float32),
            pltpu.VMEM((CHUNK,), jnp.float32),
            pltpu.VMEM((CHUNK,), jnp.float32),
            pltpu.VMEM((CHUNK,), jnp.float32),
            pltpu.SemaphoreType.DMA,
            pltpu.SemaphoreType.DMA,
        ],
        compiler_params=pltpu.CompilerParams(use_tc_tiling_on_sc=False,
                                             needs_layout_passes=False),
    )(_sc_body)
    o0, o1, o2 = fn(gene_idx, dsr, tor, table.T)
    return jnp.stack([o0, o1, o2], axis=1)


def kernel(gene_index_tensor_n, cell_index_tensor_n, downsampling_rate_tensor_n,
           total_obs_reads_per_cell_tensor_n, cell_features_nf, readout_bias_gr):
    return _sc_call(gene_index_tensor_n.astype(jnp.int32),
                    downsampling_rate_tensor_n,
                    total_obs_reads_per_cell_tensor_n,
                    readout_bias_gr)


# speedup vs baseline: 3.3601x; 1.0146x over previous
import functools

import jax
import jax.numpy as jnp
from jax import lax
from jax.experimental import pallas as pl
from jax.experimental.pallas import tpu as pltpu, tpu_sc as plsc

EPS = 1e-06
MEAN_READS = 5000.0
LN2 = 0.6931471805599453
SQRT2 = 1.4142135381698608

N_ROWS = 16384
R = 3
NC, NS, L = 2, 16, 16
NW = NC * NS
CHUNK = N_ROWS // NW
VECS = CHUNK // L


def _ln(x):
    bits = lax.bitcast_convert_type(x, jnp.int32)
    e = ((bits >> 23) & 0xFF) - 127
    m = lax.bitcast_convert_type((bits & 0x7FFFFF) | (127 << 23), jnp.float32)
    big = m > SQRT2
    m = jnp.where(big, m * 0.5, m)
    e = e + big.astype(jnp.int32)
    t = (m - 1.0) / (m + 1.0)
    z = t * t
    ln_m = 2.0 * t * (1.0 + z * (1.0 / 3.0 + z * (1.0 / 5.0 + z * (1.0 / 7.0 + z * (1.0 / 9.0)))))
    out = e.astype(jnp.float32) * LN2 + ln_m
    bad = jnp.logical_not(x < jnp.inf)
    return jnp.where(bad, x, out)


def _sc_body(gene_hbm, dsr_hbm, tor_hbm, tt_hbm, out_hbm,
             idx_v, p0_v, p1_v, p2_v, dsr_v, tor_v, sem0, sem12):
    wid = lax.axis_index("s") * NC + lax.axis_index("c")
    base = wid * CHUNK
    pltpu.sync_copy(gene_hbm.at[pl.ds(base, CHUNK)], idx_v)
    g0 = pltpu.async_copy(tt_hbm.at[0].at[idx_v], p0_v, sem0)
    g1 = pltpu.async_copy(tt_hbm.at[1].at[idx_v], p1_v, sem12)
    g2 = pltpu.async_copy(tt_hbm.at[2].at[idx_v], p2_v, sem12)
    pltpu.sync_copy(dsr_hbm.at[pl.ds(base, CHUNK)], dsr_v)
    pltpu.sync_copy(tor_hbm.at[pl.ds(base, CHUNK)], tor_v)
    g0.wait()
    for k in range(VECS):
        s = pl.ds(k * L, L)
        scale = tor_v[s] / (MEAN_READS * dsr_v[s])
        p0_v[s] = p0_v[s] + _ln(EPS + scale)
    pltpu.sync_copy(p0_v, out_hbm.at[0].at[pl.ds(base, CHUNK)])
    g1.wait()
    g2.wait()
    pltpu.sync_copy(p1_v, out_hbm.at[1].at[pl.ds(base, CHUNK)])
    pltpu.sync_copy(p2_v, out_hbm.at[2].at[pl.ds(base, CHUNK)])


@jax.jit
def _sc_call(gene_idx, dsr, tor, table):
    mesh = plsc.VectorSubcoreMesh(core_axis_name="c", subcore_axis_name="s")
    fn = functools.partial(
        pl.kernel,
        out_type=jax.ShapeDtypeStruct((R, N_ROWS), jnp.float32),
        mesh=mesh,
        scratch_types=[
            pltpu.VMEM((CHUNK,), jnp.int32),
            pltpu.VMEM((CHUNK,), jnp.float32),
            pltpu.VMEM((CHUNK,), jnp.float32),
            pltpu.VMEM((CHUNK,), jnp.float32),
            pltpu.VMEM((CHUNK,), jnp.float32),
            pltpu.VMEM((CHUNK,), jnp.float32),
            pltpu.SemaphoreType.DMA,
            pltpu.SemaphoreType.DMA,
        ],
        compiler_params=pltpu.CompilerParams(use_tc_tiling_on_sc=False,
                                             needs_layout_passes=False),
    )(_sc_body)
    return fn(gene_idx, dsr, tor, table.T).T


def kernel(gene_index_tensor_n, cell_index_tensor_n, downsampling_rate_tensor_n,
           total_obs_reads_per_cell_tensor_n, cell_features_nf, readout_bias_gr):
    return _sc_call(gene_index_tensor_n.astype(jnp.int32),
                    downsampling_rate_tensor_n,
                    total_obs_reads_per_cell_tensor_n,
                    readout_bias_gr)


# speedup vs baseline: 3.3735x; 1.0040x over previous
import functools

import jax
import jax.numpy as jnp
from jax import lax
from jax.experimental import pallas as pl
from jax.experimental.pallas import tpu as pltpu, tpu_sc as plsc

EPS = 1e-06
MEAN_READS = 5000.0
LN2 = 0.6931471805599453
SQRT2 = 1.4142135381698608

N_ROWS = 16384
R = 3
NC, NS, L = 2, 16, 16
NW = NC * NS
CHUNK = N_ROWS // NW
VECS = CHUNK // L


def _ln(x):
    bits = lax.bitcast_convert_type(x, jnp.int32)
    e = ((bits >> 23) & 0xFF) - 127
    m = lax.bitcast_convert_type((bits & 0x7FFFFF) | (127 << 23), jnp.float32)
    big = m > SQRT2
    m = jnp.where(big, m * 0.5, m)
    e = e + big.astype(jnp.int32)
    t = (m - 1.0) / (m + 1.0)
    z = t * t
    ln_m = 2.0 * t * (1.0 + z * (1.0 / 3.0 + z * (1.0 / 5.0 + z * (1.0 / 7.0 + z * (1.0 / 9.0)))))
    out = e.astype(jnp.float32) * LN2 + ln_m
    bad = jnp.logical_not(x < jnp.inf)
    return jnp.where(bad, x, out)


def _sc_body(gene_hbm, dsr_hbm, tor_hbm, tt_hbm, out_hbm,
             idx_v, p0_v, p1_v, p2_v, dsr_v, tor_v, sem_in, sem0, sem12,
             sem_out):
    wid = lax.axis_index("s") * NC + lax.axis_index("c")
    base = wid * CHUNK
    d = pltpu.async_copy(dsr_hbm.at[pl.ds(base, CHUNK)], dsr_v, sem_in)
    t = pltpu.async_copy(tor_hbm.at[pl.ds(base, CHUNK)], tor_v, sem_in)
    pltpu.sync_copy(gene_hbm.at[pl.ds(base, CHUNK)], idx_v)
    g0 = pltpu.async_copy(tt_hbm.at[0].at[idx_v], p0_v, sem0)
    g1 = pltpu.async_copy(tt_hbm.at[1].at[idx_v], p1_v, sem12)
    g2 = pltpu.async_copy(tt_hbm.at[2].at[idx_v], p2_v, sem12)
    d.wait()
    t.wait()
    g0.wait()
    for k in range(VECS):
        s = pl.ds(k * L, L)
        scale = tor_v[s] / (MEAN_READS * dsr_v[s])
        p0_v[s] = p0_v[s] + _ln(EPS + scale)
    w0 = pltpu.async_copy(p0_v, out_hbm.at[0].at[pl.ds(base, CHUNK)], sem_out)
    g1.wait()
    g2.wait()
    w1 = pltpu.async_copy(p1_v, out_hbm.at[1].at[pl.ds(base, CHUNK)], sem_out)
    w2 = pltpu.async_copy(p2_v, out_hbm.at[2].at[pl.ds(base, CHUNK)], sem_out)
    w0.wait()
    w1.wait()
    w2.wait()


@jax.jit
def _sc_call(gene_idx, dsr, tor, table):
    mesh = plsc.VectorSubcoreMesh(core_axis_name="c", subcore_axis_name="s")
    fn = functools.partial(
        pl.kernel,
        out_type=jax.ShapeDtypeStruct((R, N_ROWS), jnp.float32),
        mesh=mesh,
        scratch_types=[
            pltpu.VMEM((CHUNK,), jnp.int32),
            pltpu.VMEM((CHUNK,), jnp.float32),
            pltpu.VMEM((CHUNK,), jnp.float32),
            pltpu.VMEM((CHUNK,), jnp.float32),
            pltpu.VMEM((CHUNK,), jnp.float32),
            pltpu.VMEM((CHUNK,), jnp.float32),
            pltpu.SemaphoreType.DMA,
            pltpu.SemaphoreType.DMA,
            pltpu.SemaphoreType.DMA,
            pltpu.SemaphoreType.DMA,
        ],
        compiler_params=pltpu.CompilerParams(use_tc_tiling_on_sc=False,
                                             needs_layout_passes=False),
    )(_sc_body)
    return fn(gene_idx, dsr, tor, table.T).T


def kernel(gene_index_tensor_n, cell_index_tensor_n, downsampling_rate_tensor_n,
           total_obs_reads_per_cell_tensor_n, cell_features_nf, readout_bias_gr):
    return _sc_call(gene_index_tensor_n.astype(jnp.int32),
                    downsampling_rate_tensor_n,
                    total_obs_reads_per_cell_tensor_n,
                    readout_bias_gr)
